# Initial kernel scaffold; baseline (speedup 1.0000x reference)
import jax
import jax.numpy as jnp
from jax import lax
from jax.experimental import pallas as pl

_K_TOP = 32
_BISECT_ITERS = 26


def _body(emb_ref, xbin_ref, wbinT_ref, wregT_ref, breg_ref, out_ref):
    f32 = jnp.float32
    emb = emb_ref[:, 1:, :]
    ls = jnp.mean(emb, axis=-1)
    agg = lax.dot_general(
        ls, wbinT_ref[...], (((1,), (0,)), ((), ())),
        precision=lax.Precision.HIGHEST,
        preferred_element_type=f32,
    )

    lo = jnp.min(agg, axis=1, keepdims=True)
    hi = jnp.max(agg, axis=1, keepdims=True)

    def bisect(_, carry):
        lo, hi = carry
        mid = 0.5 * (lo + hi)
        cnt = jnp.sum((agg >= mid).astype(f32), axis=1, keepdims=True)
        pred = cnt >= _K_TOP
        return jnp.where(pred, mid, lo), jnp.where(pred, hi, mid)

    lo, hi = lax.fori_loop(0, _BISECT_ITERS, bisect, (lo, hi))
    thr = lo

    mask = (agg >= thr).astype(f32)
    x_t_bin = mask * xbin_ref[...]

    rp = lax.dot_general(
        emb_ref[:, 0, :], wregT_ref[...], (((1,), (0,)), ((), ())),
        precision=lax.Precision.HIGHEST,
        preferred_element_type=f32,
    ) + breg_ref[...]

    out = rp[:, 0:1] + jnp.sum(x_t_bin * rp[:, 1:], axis=1, keepdims=True)
    out_ref[...] = out


def kernel(x_t, x_m_emb, x_bin, W_bin, W_reg, b_reg):
    B, Fp1, D = x_m_emb.shape
    BF = x_bin.shape[1]
    BB = 512
    grid = (B // BB,)

    wbinT = W_bin.T
    wregT = W_reg.T
    breg2 = b_reg.reshape(1, 1 + BF)

    out = pl.pallas_call(
        _body,
        grid=grid,
        in_specs=[
            pl.BlockSpec((BB, Fp1, D), lambda i: (i, 0, 0)),
            pl.BlockSpec((BB, BF), lambda i: (i, 0)),
            pl.BlockSpec((Fp1 - 1, BF), lambda i: (0, 0)),
            pl.BlockSpec((D, 1 + BF), lambda i: (0, 0)),
            pl.BlockSpec((1, 1 + BF), lambda i: (0, 0)),
        ],
        out_specs=pl.BlockSpec((BB, 1), lambda i: (i, 0)),
        out_shape=jax.ShapeDtypeStruct((B, 1), jnp.float32),
    )(x_m_emb, x_bin, wbinT, wregT, breg2)
    return out


# speedup vs baseline: 4.8573x; 4.8573x over previous
import jax
import jax.numpy as jnp
from jax import lax
from jax.experimental import pallas as pl

_K_TOP = 32
_BISECT_ITERS = 26


def _body(emb_ref, xbin_ref, wbinT_ref, wregT_ref, breg_ref, out_ref):
    f32 = jnp.float32
    emb = emb_ref[:, 1:, :]
    ls = jnp.mean(emb, axis=-1)
    agg = lax.dot_general(
        ls, wbinT_ref[...], (((1,), (0,)), ((), ())),
        preferred_element_type=f32,
    )

    lo = jnp.min(agg, axis=1, keepdims=True)
    hi = jnp.max(agg, axis=1, keepdims=True)

    def bisect(_, carry):
        lo, hi = carry
        mid = 0.5 * (lo + hi)
        cnt = jnp.sum((agg >= mid).astype(f32), axis=1, keepdims=True)
        pred = cnt >= _K_TOP
        return jnp.where(pred, mid, lo), jnp.where(pred, hi, mid)

    lo, hi = lax.fori_loop(0, _BISECT_ITERS, bisect, (lo, hi))
    thr = lo

    mask = (agg >= thr).astype(f32)
    x_t_bin = mask * xbin_ref[...]

    rp = lax.dot_general(
        emb_ref[:, 0, :], wregT_ref[...], (((1,), (0,)), ((), ())),
        preferred_element_type=f32,
    ) + breg_ref[...]

    out = rp[:, 0:1] + jnp.sum(x_t_bin * rp[:, 1:], axis=1, keepdims=True)
    out_ref[...] = out


def kernel(x_t, x_m_emb, x_bin, W_bin, W_reg, b_reg):
    B, Fp1, D = x_m_emb.shape
    BF = x_bin.shape[1]
    BB = 512
    grid = (B // BB,)

    wbinT = W_bin.T
    wregT = W_reg.T
    breg2 = b_reg.reshape(1, 1 + BF)

    out = pl.pallas_call(
        _body,
        grid=grid,
        in_specs=[
            pl.BlockSpec((BB, Fp1, D), lambda i: (i, 0, 0)),
            pl.BlockSpec((BB, BF), lambda i: (i, 0)),
            pl.BlockSpec((Fp1 - 1, BF), lambda i: (0, 0)),
            pl.BlockSpec((D, 1 + BF), lambda i: (0, 0)),
            pl.BlockSpec((1, 1 + BF), lambda i: (0, 0)),
        ],
        out_specs=pl.BlockSpec((BB, 1), lambda i: (i, 0)),
        out_shape=jax.ShapeDtypeStruct((B, 1), jnp.float32),
    )(x_m_emb, x_bin, wbinT, wregT, breg2)
    return out
